# Initial kernel scaffold; baseline (speedup 1.0000x reference)
import jax
import jax.numpy as jnp
from jax.experimental import pallas as pl


def kernel(x, edge_index, fair_node_embedding, W1, a_src1, a_dst1, b1, gamma1, beta1, W2, a_src2, a_dst2, b2, gamma2, beta2, lin1_W, lin1_b, lin2_W, lin2_b):
    raise NotImplementedError("write your pallas kernel here")


# speedup vs baseline: 16.2093x; 16.2093x over previous
import functools

import jax
import jax.numpy as jnp
from jax import lax
from jax.experimental import pallas as pl
from jax.experimental.pallas import tpu as pltpu
from jax.experimental.pallas import tpu_sc as plsc

N = 10000
E = 320000
D = 128
QD = 32
EPT = E // 16
CH = 80
NCH = EPT // CH
NR = 10240
RPS = NR // 16
BR = 2000
GRID = N // BR

_NEG = -3.0e38


def _dense_body(with_bn, *refs):
    if with_bn:
        (x_ref, mu_ref, rstd_ref, gam_ref, bet_ref, w_ref, as_ref, ad_ref,
         h0_ref, h1_ref, h2_ref, h3_ref, als_ref, ald_ref, mxs_ref,
         mxd_ref) = refs
        xb = x_ref[...]
        xb = gam_ref[...] * (xb - mu_ref[...]) * rstd_ref[...] + bet_ref[...]
        xb = jnp.maximum(xb, 0.0)
    else:
        (x_ref, w_ref, as_ref, ad_ref,
         h0_ref, h1_ref, h2_ref, h3_ref, als_ref, ald_ref, mxs_ref,
         mxd_ref) = refs
        xb = x_ref[...]
    i = pl.program_id(0)
    h = jnp.dot(xb, w_ref[...], preferred_element_type=jnp.float32)
    h0_ref[...] = h[:, 0 * QD:1 * QD]
    h1_ref[...] = h[:, 1 * QD:2 * QD]
    h2_ref[...] = h[:, 2 * QD:3 * QD]
    h3_ref[...] = h[:, 3 * QD:4 * QD]
    als = jnp.sum(h * as_ref[...], axis=1, keepdims=True)
    ald = jnp.sum(h * ad_ref[...], axis=1, keepdims=True)
    als_ref[...] = als
    ald_ref[...] = ald

    @pl.when(i == 0)
    def _():
        mxs_ref[...] = jnp.full((1, D), _NEG, jnp.float32)
        mxd_ref[...] = jnp.full((1, D), _NEG, jnp.float32)

    mxs_ref[...] = jnp.maximum(mxs_ref[...], jnp.max(als))
    mxd_ref[...] = jnp.maximum(mxd_ref[...], jnp.max(ald))


def _dense_call(with_bn, x, W, a_s, a_d, bn=None):
    row_spec = pl.BlockSpec((BR, D), lambda i: (i, 0))
    q_spec = pl.BlockSpec((BR, QD), lambda i: (i, 0))
    full_spec = pl.BlockSpec((1, D), lambda i: (0, 0))
    in_specs = [row_spec]
    args = [x]
    if with_bn:
        mu, rstd, gam, bet = bn
        in_specs += [full_spec] * 4
        args += [mu, rstd, gam, bet]
    in_specs += [pl.BlockSpec((D, D), lambda i: (0, 0)), full_spec, full_spec]
    args += [W, a_s, a_d]
    return pl.pallas_call(
        functools.partial(_dense_body, with_bn),
        grid=(GRID,),
        in_specs=in_specs,
        out_specs=[
            q_spec, q_spec, q_spec, q_spec,
            pl.BlockSpec((BR, 1), lambda i: (i, 0)),
            pl.BlockSpec((BR, 1), lambda i: (i, 0)),
            full_spec,
            full_spec,
        ],
        out_shape=[
            jax.ShapeDtypeStruct((N, QD), jnp.float32),
            jax.ShapeDtypeStruct((N, QD), jnp.float32),
            jax.ShapeDtypeStruct((N, QD), jnp.float32),
            jax.ShapeDtypeStruct((N, QD), jnp.float32),
            jax.ShapeDtypeStruct((N, 1), jnp.float32),
            jax.ShapeDtypeStruct((N, 1), jnp.float32),
            jax.ShapeDtypeStruct((1, D), jnp.float32),
            jax.ShapeDtypeStruct((1, D), jnp.float32),
        ],
    )(*args)


_sc_mesh = plsc.VectorSubcoreMesh(core_axis_name="c", subcore_axis_name="s")


@functools.partial(
    pl.kernel,
    mesh=_sc_mesh,
    compiler_params=pltpu.CompilerParams(needs_layout_passes=False,
                                         use_tc_tiling_on_sc=False),
    out_type=[
        jax.ShapeDtypeStruct((4, NR, QD), jnp.float32),
        jax.ShapeDtypeStruct((NR,), jnp.float32),
    ],
    scratch_types=[
        pltpu.VMEM((NCH, CH), jnp.int32),
        pltpu.VMEM((NCH, CH), jnp.int32),
        pltpu.VMEM((N,), jnp.float32),
        pltpu.VMEM((N,), jnp.float32),
        pltpu.VMEM((16,), jnp.float32),
        pltpu.VMEM((NCH, CH), jnp.float32),
        pltpu.VMEM((CH, QD), jnp.float32),
        pltpu.VMEM((RPS // 5, QD), jnp.float32),
        pltpu.VMEM((RPS,), jnp.float32),
        pltpu.VMEM_SHARED((NR, QD), jnp.float32),
        pltpu.VMEM_SHARED((NR,), jnp.float32),
        pltpu.SemaphoreType.DMA,
    ],
)
def _edge_kernel(edge_hbm, als_hbm, ald_hbm, m_hbm, h0_hbm, h1_hbm, h2_hbm,
                 h3_hbm, out_hbm, den_hbm,
                 src_v, dst_v, als_v, ald_v, m_v, w_v, rows_v, zrow_v, zden_v,
                 out_sh, den_sh, sem):
    c = lax.axis_index("c")
    s = lax.axis_index("s")

    pltpu.sync_copy(edge_hbm.at[0, s], src_v)
    pltpu.sync_copy(edge_hbm.at[1, s], dst_v)
    pltpu.sync_copy(als_hbm, als_v)
    pltpu.sync_copy(ald_hbm, ald_v)
    pltpu.sync_copy(m_hbm, m_v)

    zv = jnp.zeros((16,), jnp.float32)

    def _zrow(r, carry):
        for t in range(QD // 16):
            zrow_v[r, pl.ds(t * 16, 16)] = zv
        return carry

    lax.fori_loop(0, RPS // 5, _zrow, 0)
    for g in range(RPS // 16):
        zden_v[pl.ds(g * 16, 16)] = zv

    def _zero_stripe():
        for q in range(5):
            pltpu.sync_copy(
                zrow_v, out_sh.at[pl.ds(s * RPS + q * (RPS // 5), RPS // 5)])

    _zero_stripe()
    pltpu.sync_copy(zden_v, den_sh.at[pl.ds(s * RPS, RPS)])
    plsc.subcore_barrier()

    m16 = m_v[...]

    def _make_chunk(first_pass):
        def _chunk(j, carry):
            if first_pass:
                for v in range(CH // 16):
                    si = src_v[j, pl.ds(v * 16, 16)]
                    di = dst_v[j, pl.ds(v * 16, 16)]
                    a1 = plsc.load_gather(als_v, [si])
                    a2 = plsc.load_gather(ald_v, [di])
                    z = a1 + a2
                    z = jnp.where(z > 0.0, z, 0.2 * z)
                    w_v[j, pl.ds(v * 16, 16)] = jnp.exp(z - m16)
            @pl.when(c == 0)
            def _():
                tab = h0_hbm if first_pass else h1_hbm
                pltpu.async_copy(tab.at[src_v.at[j]], rows_v, sem).wait()

            @pl.when(c == 1)
            def _():
                tab = h2_hbm if first_pass else h3_hbm
                pltpu.async_copy(tab.at[src_v.at[j]], rows_v, sem).wait()

            for v in range(CH // 16):
                wg = w_v[j, pl.ds(v * 16, 16)]
                for i in range(16):
                    ws = wg[i]
                    r = v * 16 + i
                    for t in range(QD // 16):
                        rows_v[r, pl.ds(t * 16, 16)] = (
                            rows_v[r, pl.ds(t * 16, 16)] * ws)
            pltpu.sync_copy(rows_v, out_sh.at[dst_v.at[j]], add=True)
            if first_pass:
                pltpu.sync_copy(w_v.at[j], den_sh.at[dst_v.at[j]], add=True)
            return carry
        return _chunk

    lax.fori_loop(0, NCH, _make_chunk(True), 0)
    plsc.subcore_barrier()
    for qq in range(2):
        @pl.when(c == qq)
        def _():
            pltpu.sync_copy(out_sh.at[pl.ds(s * RPS, RPS)],
                            out_hbm.at[2 * qq, pl.ds(s * RPS, RPS)])
    @pl.when(c == 0)
    def _():
        pltpu.sync_copy(den_sh.at[pl.ds(s * RPS, RPS)],
                        den_hbm.at[pl.ds(s * RPS, RPS)])
    _zero_stripe()
    plsc.subcore_barrier()

    lax.fori_loop(0, NCH, _make_chunk(False), 0)
    plsc.subcore_barrier()
    for qq in range(2):
        @pl.when(c == qq)
        def _():
            pltpu.sync_copy(out_sh.at[pl.ds(s * RPS, RPS)],
                            out_hbm.at[2 * qq + 1, pl.ds(s * RPS, RPS)])


def _norm_body(p_ref, d_ref, h0_ref, h1_ref, h2_ref, h3_ref, als_ref,
               ald_ref, m_ref, b_ref, g_ref, sum_ref, sq_ref):
    i = pl.program_id(0)
    m = m_ref[0, 0]
    z = als_ref[...] + ald_ref[...]
    z = jnp.where(z > 0.0, z, 0.2 * z)
    ws = jnp.exp(z - m)
    h = jnp.concatenate(
        [h0_ref[...], h1_ref[...], h2_ref[...], h3_ref[...]], axis=1)
    p = jnp.concatenate(
        [p_ref[0], p_ref[1], p_ref[2], p_ref[3]], axis=1)
    num = p + ws * h
    den = d_ref[...] + ws
    g = num / (den + 1e-16) + b_ref[...]
    g_ref[...] = g

    @pl.when(i == 0)
    def _():
        sum_ref[...] = jnp.zeros((1, D), jnp.float32)
        sq_ref[...] = jnp.zeros((1, D), jnp.float32)

    sum_ref[...] += jnp.sum(g, axis=0, keepdims=True)
    sq_ref[...] += jnp.sum(g * g, axis=0, keepdims=True)


def _norm_call(p, d2, hq, als, ald, m_row, bias):
    return pl.pallas_call(
        _norm_body,
        grid=(GRID,),
        in_specs=[
            pl.BlockSpec((4, BR, QD), lambda i: (0, i, 0)),
            pl.BlockSpec((BR, 1), lambda i: (i, 0)),
            pl.BlockSpec((BR, QD), lambda i: (i, 0)),
            pl.BlockSpec((BR, QD), lambda i: (i, 0)),
            pl.BlockSpec((BR, QD), lambda i: (i, 0)),
            pl.BlockSpec((BR, QD), lambda i: (i, 0)),
            pl.BlockSpec((BR, 1), lambda i: (i, 0)),
            pl.BlockSpec((BR, 1), lambda i: (i, 0)),
            pl.BlockSpec((1, D), lambda i: (0, 0)),
            pl.BlockSpec((1, D), lambda i: (0, 0)),
        ],
        out_specs=[
            pl.BlockSpec((BR, D), lambda i: (i, 0)),
            pl.BlockSpec((1, D), lambda i: (0, 0)),
            pl.BlockSpec((1, D), lambda i: (0, 0)),
        ],
        out_shape=[
            jax.ShapeDtypeStruct((N, D), jnp.float32),
            jax.ShapeDtypeStruct((1, D), jnp.float32),
            jax.ShapeDtypeStruct((1, D), jnp.float32),
        ],
    )(p, d2, hq[0], hq[1], hq[2], hq[3], als, ald, m_row, bias)


def _head_body(g_ref, mu_ref, rstd_ref, gam_ref, bet_ref, emb_ref,
               w1a_ref, w1b_ref, b1_ref, w2_ref, b2_ref, out_ref):
    y = gam_ref[...] * (g_ref[...] - mu_ref[...]) * rstd_ref[...] + bet_ref[...]
    y = jnp.maximum(y, 0.0)
    z = jnp.dot(y, w1a_ref[...], preferred_element_type=jnp.float32)
    z += jnp.dot(emb_ref[...], w1b_ref[...], preferred_element_type=jnp.float32)
    z = jnp.maximum(z + b1_ref[...], 0.0)
    out_ref[...] = jnp.dot(z, w2_ref[...],
                           preferred_element_type=jnp.float32) + b2_ref[...]


def _head_call(g, mu, rstd, gam, bet, emb, w1a, w1b, b1, w2, b2):
    full = pl.BlockSpec((1, D), lambda i: (0, 0))
    return pl.pallas_call(
        _head_body,
        grid=(GRID,),
        in_specs=[
            pl.BlockSpec((BR, D), lambda i: (i, 0)),
            full, full, full, full,
            pl.BlockSpec((BR, 64), lambda i: (i, 0)),
            pl.BlockSpec((D, D), lambda i: (0, 0)),
            pl.BlockSpec((64, D), lambda i: (0, 0)),
            full,
            pl.BlockSpec((D, 2), lambda i: (0, 0)),
            pl.BlockSpec((1, 2), lambda i: (0, 0)),
        ],
        out_specs=pl.BlockSpec((BR, 2), lambda i: (i, 0)),
        out_shape=jax.ShapeDtypeStruct((N, 2), jnp.float32),
    )(g, mu, rstd, gam, bet, emb, w1a, w1b, b1, w2, b2)


def _softmax_shift(mxs, mxd):
    b = mxs + mxd
    mrow = jnp.where(b > 0.0, b, 0.2 * b)
    return mrow, mrow[0, :16]


def _moments(ssum, ssq):
    mu = ssum / N
    var = ssq / N - mu * mu
    rstd = 1.0 / jnp.sqrt(var + 1e-5)
    return mu, rstd


def _gat_layer(edge_r, x, W, a_s, a_d, bias, bn):
    h0, h1, h2, h3, als, ald, mxs, mxd = _dense_call(bn is not None, x, W,
                                                     a_s, a_d, bn)
    mrow, m16 = _softmax_shift(mxs, mxd)
    outp, denp = _edge_kernel(edge_r, als.reshape(N), ald.reshape(N), m16,
                              h0, h1, h2, h3)
    d2 = denp.reshape(NR, 1)
    g, ssum, ssq = _norm_call(outp, d2, (h0, h1, h2, h3), als, ald, mrow,
                              bias)
    mu, rstd = _moments(ssum, ssq)
    return g, mu, rstd


def kernel(x, edge_index, fair_node_embedding, W1, a_src1, a_dst1, b1,
           gamma1, beta1, W2, a_src2, a_dst2, b2, gamma2, beta2,
           lin1_W, lin1_b, lin2_W, lin2_b):
    edge_r = edge_index.reshape(2, 16, NCH, CH)
    row = lambda v: v.reshape(1, -1)

    g1, mu1, rstd1 = _gat_layer(edge_r, x, W1, a_src1.reshape(1, D),
                                a_dst1.reshape(1, D), row(b1), None)
    g2, mu2, rstd2 = _gat_layer(edge_r, g1, W2, a_src2.reshape(1, D),
                                a_dst2.reshape(1, D), row(b2),
                                (mu1, rstd1, row(gamma1), row(beta1)))
    return _head_call(g2, mu2, rstd2, row(gamma2), row(beta2),
                      fair_node_embedding, lin1_W[:D], lin1_W[D:],
                      row(lin1_b), lin2_W, row(lin2_b))


# speedup vs baseline: 36.1859x; 2.2324x over previous
import functools

import jax
import jax.numpy as jnp
from jax import lax
from jax.experimental import pallas as pl
from jax.experimental.pallas import tpu as pltpu
from jax.experimental.pallas import tpu_sc as plsc

N = 10000
E = 320000
D = 128
QD = 32
EPT = E // 16
CH = 80
NCH = EPT // CH
NR = 10240
RPS = NR // 16
BR = 2000
GRID = N // BR

_NEG = -3.0e38


def _dense_body(with_bn, *refs):
    if with_bn:
        (x_ref, sum_ref, sq_ref, gam_ref, bet_ref, w_ref, as_ref, ad_ref,
         h0_ref, h1_ref, h2_ref, h3_ref, als_ref, ald_ref, mxs_ref,
         mxd_ref) = refs
        mu = sum_ref[...] * (1.0 / N)
        var = sq_ref[...] * (1.0 / N) - mu * mu
        rstd = 1.0 / jnp.sqrt(var + 1e-5)
        xb = x_ref[...]
        xb = gam_ref[...] * (xb - mu) * rstd + bet_ref[...]
        xb = jnp.maximum(xb, 0.0)
    else:
        (x_ref, w_ref, as_ref, ad_ref,
         h0_ref, h1_ref, h2_ref, h3_ref, als_ref, ald_ref, mxs_ref,
         mxd_ref) = refs
        xb = x_ref[...]
    i = pl.program_id(0)
    h = jnp.dot(xb, w_ref[...], preferred_element_type=jnp.float32)
    h0_ref[...] = h[:, 0 * QD:1 * QD]
    h1_ref[...] = h[:, 1 * QD:2 * QD]
    h2_ref[...] = h[:, 2 * QD:3 * QD]
    h3_ref[...] = h[:, 3 * QD:4 * QD]
    als = jnp.sum(h * as_ref[...], axis=1, keepdims=True)
    ald = jnp.sum(h * ad_ref[...], axis=1, keepdims=True)
    als_ref[...] = als
    ald_ref[...] = ald

    @pl.when(i == 0)
    def _():
        mxs_ref[...] = jnp.full((1, D), _NEG, jnp.float32)
        mxd_ref[...] = jnp.full((1, D), _NEG, jnp.float32)

    mxs_ref[...] = jnp.maximum(mxs_ref[...], jnp.max(als))
    mxd_ref[...] = jnp.maximum(mxd_ref[...], jnp.max(ald))


def _dense_call(with_bn, x, W, a_s, a_d, bn=None):
    row_spec = pl.BlockSpec((BR, D), lambda i: (i, 0))
    q_spec = pl.BlockSpec((BR, QD), lambda i: (i, 0))
    full_spec = pl.BlockSpec((1, D), lambda i: (0, 0))
    in_specs = [row_spec]
    args = [x]
    if with_bn:
        ssum, ssq, gam, bet = bn
        in_specs += [full_spec] * 4
        args += [ssum, ssq, gam, bet]
    in_specs += [pl.BlockSpec((D, D), lambda i: (0, 0)), full_spec, full_spec]
    args += [W, a_s, a_d]
    return pl.pallas_call(
        functools.partial(_dense_body, with_bn),
        grid=(GRID,),
        in_specs=in_specs,
        out_specs=[
            q_spec, q_spec, q_spec, q_spec,
            pl.BlockSpec((BR, 1), lambda i: (i, 0)),
            pl.BlockSpec((BR, 1), lambda i: (i, 0)),
            full_spec,
            full_spec,
        ],
        out_shape=[
            jax.ShapeDtypeStruct((N, QD), jnp.float32),
            jax.ShapeDtypeStruct((N, QD), jnp.float32),
            jax.ShapeDtypeStruct((N, QD), jnp.float32),
            jax.ShapeDtypeStruct((N, QD), jnp.float32),
            jax.ShapeDtypeStruct((N, 1), jnp.float32),
            jax.ShapeDtypeStruct((N, 1), jnp.float32),
            jax.ShapeDtypeStruct((1, D), jnp.float32),
            jax.ShapeDtypeStruct((1, D), jnp.float32),
        ],
    )(*args)


_sc_mesh = plsc.VectorSubcoreMesh(core_axis_name="c", subcore_axis_name="s")


@functools.partial(
    pl.kernel,
    mesh=_sc_mesh,
    compiler_params=pltpu.CompilerParams(needs_layout_passes=False,
                                         use_tc_tiling_on_sc=False),
    out_type=[
        jax.ShapeDtypeStruct((4, NR, QD), jnp.float32),
        jax.ShapeDtypeStruct((NR,), jnp.float32),
    ],
    scratch_types=[
        pltpu.VMEM((NCH, CH), jnp.int32),
        pltpu.VMEM((NCH, CH), jnp.int32),
        pltpu.VMEM((N,), jnp.float32),
        pltpu.VMEM((N,), jnp.float32),
        pltpu.VMEM((16,), jnp.float32),
        pltpu.VMEM((16,), jnp.float32),
        pltpu.VMEM((NCH, CH), jnp.float32),
        pltpu.VMEM((CH, QD), jnp.float32),
        pltpu.VMEM((CH, QD), jnp.float32),
        pltpu.VMEM((CH, QD), jnp.float32),
        pltpu.VMEM((CH, QD), jnp.float32),
        pltpu.VMEM((CH, QD), jnp.float32),
        pltpu.VMEM((RPS // 5, QD), jnp.float32),
        pltpu.VMEM((RPS,), jnp.float32),
        pltpu.VMEM_SHARED((NR, QD), jnp.float32),
        pltpu.VMEM_SHARED((NR,), jnp.float32),
        pltpu.SemaphoreType.DMA,
        pltpu.SemaphoreType.DMA,
        pltpu.SemaphoreType.DMA,
        pltpu.SemaphoreType.DMA,
        pltpu.SemaphoreType.DMA,
        pltpu.SemaphoreType.DMA,
        pltpu.SemaphoreType.DMA,
        pltpu.SemaphoreType.DMA,
        pltpu.SemaphoreType.DMA,
        pltpu.SemaphoreType.DMA,
        pltpu.SemaphoreType.DMA,
    ],
)
def _edge_kernel(edge_hbm, als_hbm, ald_hbm, mxs_hbm, mxd_hbm, h0_hbm,
                 h1_hbm, h2_hbm, h3_hbm, out_hbm, den_hbm,
                 src_v, dst_v, als_v, ald_v, mxs_v, mxd_v, w_v,
                 rows0, rows1, rows2, rows3, rows4, zrow_v, zden_v,
                 out_sh, den_sh,
                 gs0, gs1, gs2, gs3, gs4, ss0, ss1, ss2, ss3, ss4, dsem):
    c = lax.axis_index("c")
    s = lax.axis_index("s")
    rows = [rows0, rows1, rows2, rows3, rows4]
    gsem = [gs0, gs1, gs2, gs3, gs4]
    ssem = [ss0, ss1, ss2, ss3, ss4]

    pltpu.sync_copy(edge_hbm.at[0, s], src_v)
    pltpu.sync_copy(edge_hbm.at[1, s], dst_v)
    pltpu.sync_copy(als_hbm, als_v)
    pltpu.sync_copy(ald_hbm, ald_v)
    pltpu.sync_copy(mxs_hbm.at[0, pl.ds(0, 16)], mxs_v)
    pltpu.sync_copy(mxd_hbm.at[0, pl.ds(0, 16)], mxd_v)

    zv = jnp.zeros((16,), jnp.float32)

    def _zrow(r, carry):
        for t in range(QD // 16):
            zrow_v[r, pl.ds(t * 16, 16)] = zv
        return carry

    lax.fori_loop(0, RPS // 5, _zrow, 0)
    for g in range(RPS // 16):
        zden_v[pl.ds(g * 16, 16)] = zv

    def _zero_stripe():
        for q in range(5):
            pltpu.sync_copy(
                zrow_v, out_sh.at[pl.ds(s * RPS + q * (RPS // 5), RPS // 5)])

    _zero_stripe()
    pltpu.sync_copy(zden_v, den_sh.at[pl.ds(s * RPS, RPS)])
    plsc.subcore_barrier()

    mb = mxs_v[...] + mxd_v[...]
    m16 = jnp.where(mb > 0.0, mb, 0.2 * mb)

    def _wloop(j, carry):
        for v in range(CH // 16):
            si = src_v[j, pl.ds(v * 16, 16)]
            di = dst_v[j, pl.ds(v * 16, 16)]
            a1 = plsc.load_gather(als_v, [si])
            a2 = plsc.load_gather(ald_v, [di])
            z = a1 + a2
            z = jnp.where(z > 0.0, z, 0.2 * z)
            w_v[j, pl.ds(v * 16, 16)] = jnp.exp(z - m16)
        pltpu.async_copy(w_v.at[j], den_sh.at[dst_v.at[j]], dsem, add=True)
        return carry

    lax.fori_loop(0, NCH, _wloop, 0)

    def _issue_gather(j, b, first_pass):
        @pl.when(c == 0)
        def _():
            tab = h0_hbm if first_pass else h1_hbm
            pltpu.async_copy(tab.at[src_v.at[j]], rows[b], gsem[b])

        @pl.when(c == 1)
        def _():
            tab = h2_hbm if first_pass else h3_hbm
            pltpu.async_copy(tab.at[src_v.at[j]], rows[b], gsem[b])

    def _gather_wait(j, b, first_pass):
        @pl.when(c == 0)
        def _():
            tab = h0_hbm if first_pass else h1_hbm
            pltpu.make_async_copy(tab.at[src_v.at[j]], rows[b],
                                  gsem[b]).wait()

        @pl.when(c == 1)
        def _():
            tab = h2_hbm if first_pass else h3_hbm
            pltpu.make_async_copy(tab.at[src_v.at[j]], rows[b],
                                  gsem[b]).wait()

    def _scatter_wait(j, b):
        pltpu.make_async_copy(rows[b], out_sh.at[dst_v.at[j]],
                              ssem[b]).wait()

    def _scale(j, b):
        for v in range(CH // 16):
            wg = w_v[j, pl.ds(v * 16, 16)]
            for i in range(16):
                ws = wg[i]
                r = v * 16 + i
                for t in range(QD // 16):
                    rows[b][r, pl.ds(t * 16, 16)] = (
                        rows[b][r, pl.ds(t * 16, 16)] * ws)

    def _run_pass(first_pass):
        for b in range(3):
            _issue_gather(b, b, first_pass)

        def _outer(t, carry):
            for b in range(5):
                j = t * 5 + b
                _gather_wait(j, b, first_pass)
                _scale(j, b)
                pltpu.async_copy(rows[b], out_sh.at[dst_v.at[j]], ssem[b],
                                 add=True)
                b3 = (b + 3) % 5

                @pl.when(j + 3 < NCH)
                def _():
                    @pl.when(j >= 2)
                    def _():
                        _scatter_wait(j - 2, b3)
                    _issue_gather(j + 3, b3, first_pass)
            return carry

        lax.fori_loop(0, NCH // 5, _outer, 0)
        for b in range(5):
            _scatter_wait(NCH - 5 + b, b)

    _run_pass(True)
    def _den_drain(j, carry):
        pltpu.make_async_copy(w_v.at[j], den_sh.at[dst_v.at[j]], dsem).wait()
        return carry

    lax.fori_loop(0, NCH, _den_drain, 0)
    plsc.subcore_barrier()
    for qq in range(2):
        @pl.when(c == qq)
        def _():
            pltpu.sync_copy(out_sh.at[pl.ds(s * RPS, RPS)],
                            out_hbm.at[2 * qq, pl.ds(s * RPS, RPS)])
    @pl.when(c == 0)
    def _():
        pltpu.sync_copy(den_sh.at[pl.ds(s * RPS, RPS)],
                        den_hbm.at[pl.ds(s * RPS, RPS)])
    _zero_stripe()
    plsc.subcore_barrier()

    _run_pass(False)
    plsc.subcore_barrier()
    for qq in range(2):
        @pl.when(c == qq)
        def _():
            pltpu.sync_copy(out_sh.at[pl.ds(s * RPS, RPS)],
                            out_hbm.at[2 * qq + 1, pl.ds(s * RPS, RPS)])


def _norm_body(p_ref, d_ref, h0_ref, h1_ref, h2_ref, h3_ref, als_ref,
               ald_ref, mxs_ref, mxd_ref, b_ref, g_ref, sum_ref, sq_ref):
    i = pl.program_id(0)
    mb = mxs_ref[0, 0] + mxd_ref[0, 0]
    m = jnp.where(mb > 0.0, mb, 0.2 * mb)
    z = als_ref[...] + ald_ref[...]
    z = jnp.where(z > 0.0, z, 0.2 * z)
    ws = jnp.exp(z - m)
    h = jnp.concatenate(
        [h0_ref[...], h1_ref[...], h2_ref[...], h3_ref[...]], axis=1)
    p = jnp.concatenate(
        [p_ref[0], p_ref[1], p_ref[2], p_ref[3]], axis=1)
    num = p + ws * h
    den = d_ref[...] + ws
    g = num / (den + 1e-16) + b_ref[...]
    g_ref[...] = g

    @pl.when(i == 0)
    def _():
        sum_ref[...] = jnp.zeros((1, D), jnp.float32)
        sq_ref[...] = jnp.zeros((1, D), jnp.float32)

    sum_ref[...] += jnp.sum(g, axis=0, keepdims=True)
    sq_ref[...] += jnp.sum(g * g, axis=0, keepdims=True)


def _norm_call(p, d2, hq, als, ald, mxs, mxd, bias):
    return pl.pallas_call(
        _norm_body,
        grid=(GRID,),
        in_specs=[
            pl.BlockSpec((4, BR, QD), lambda i: (0, i, 0)),
            pl.BlockSpec((BR, 1), lambda i: (i, 0)),
            pl.BlockSpec((BR, QD), lambda i: (i, 0)),
            pl.BlockSpec((BR, QD), lambda i: (i, 0)),
            pl.BlockSpec((BR, QD), lambda i: (i, 0)),
            pl.BlockSpec((BR, QD), lambda i: (i, 0)),
            pl.BlockSpec((BR, 1), lambda i: (i, 0)),
            pl.BlockSpec((BR, 1), lambda i: (i, 0)),
            pl.BlockSpec((1, D), lambda i: (0, 0)),
            pl.BlockSpec((1, D), lambda i: (0, 0)),
            pl.BlockSpec((1, D), lambda i: (0, 0)),
        ],
        out_specs=[
            pl.BlockSpec((BR, D), lambda i: (i, 0)),
            pl.BlockSpec((1, D), lambda i: (0, 0)),
            pl.BlockSpec((1, D), lambda i: (0, 0)),
        ],
        out_shape=[
            jax.ShapeDtypeStruct((N, D), jnp.float32),
            jax.ShapeDtypeStruct((1, D), jnp.float32),
            jax.ShapeDtypeStruct((1, D), jnp.float32),
        ],
    )(p, d2, hq[0], hq[1], hq[2], hq[3], als, ald, mxs, mxd, bias)


def _head_body(g_ref, sum_ref, sq_ref, gam_ref, bet_ref, emb_ref,
               w1a_ref, w1b_ref, b1_ref, w2_ref, b2_ref, out_ref):
    mu = sum_ref[...] * (1.0 / N)
    var = sq_ref[...] * (1.0 / N) - mu * mu
    rstd = 1.0 / jnp.sqrt(var + 1e-5)
    y = gam_ref[...] * (g_ref[...] - mu) * rstd + bet_ref[...]
    y = jnp.maximum(y, 0.0)
    z = jnp.dot(y, w1a_ref[...], preferred_element_type=jnp.float32)
    z += jnp.dot(emb_ref[...], w1b_ref[...], preferred_element_type=jnp.float32)
    z = jnp.maximum(z + b1_ref[...], 0.0)
    out_ref[...] = jnp.dot(z, w2_ref[...],
                           preferred_element_type=jnp.float32) + b2_ref[...]


def _head_call(g, ssum, ssq, gam, bet, emb, w1a, w1b, b1, w2, b2):
    full = pl.BlockSpec((1, D), lambda i: (0, 0))
    return pl.pallas_call(
        _head_body,
        grid=(GRID,),
        in_specs=[
            pl.BlockSpec((BR, D), lambda i: (i, 0)),
            full, full, full, full,
            pl.BlockSpec((BR, 64), lambda i: (i, 0)),
            pl.BlockSpec((D, D), lambda i: (0, 0)),
            pl.BlockSpec((64, D), lambda i: (0, 0)),
            full,
            pl.BlockSpec((D, 2), lambda i: (0, 0)),
            pl.BlockSpec((1, 2), lambda i: (0, 0)),
        ],
        out_specs=pl.BlockSpec((BR, 2), lambda i: (i, 0)),
        out_shape=jax.ShapeDtypeStruct((N, 2), jnp.float32),
    )(g, ssum, ssq, gam, bet, emb, w1a, w1b, b1, w2, b2)


def _gat_layer(edge_r, x, W, a_s, a_d, bias, bn):
    h0, h1, h2, h3, als, ald, mxs, mxd = _dense_call(bn is not None, x, W,
                                                     a_s, a_d, bn)
    outp, denp = _edge_kernel(edge_r, als.reshape(N), ald.reshape(N), mxs,
                              mxd, h0, h1, h2, h3)
    d2 = denp.reshape(NR, 1)
    g, ssum, ssq = _norm_call(outp, d2, (h0, h1, h2, h3), als, ald, mxs,
                              mxd, bias)
    return g, ssum, ssq


def kernel(x, edge_index, fair_node_embedding, W1, a_src1, a_dst1, b1,
           gamma1, beta1, W2, a_src2, a_dst2, b2, gamma2, beta2,
           lin1_W, lin1_b, lin2_W, lin2_b):
    edge_r = edge_index.reshape(2, 16, NCH, CH)
    row = lambda v: v.reshape(1, -1)

    g1, ssum1, ssq1 = _gat_layer(edge_r, x, W1, a_src1.reshape(1, D),
                                 a_dst1.reshape(1, D), row(b1), None)
    g2, ssum2, ssq2 = _gat_layer(edge_r, g1, W2, a_src2.reshape(1, D),
                                 a_dst2.reshape(1, D), row(b2),
                                 (ssum1, ssq1, row(gamma1), row(beta1)))
    return _head_call(g2, ssum2, ssq2, row(gamma2), row(beta2),
                      fair_node_embedding, lin1_W[:D], lin1_W[D:],
                      row(lin1_b), lin2_W, row(lin2_b))


# speedup vs baseline: 36.2600x; 1.0020x over previous
import functools

import jax
import jax.numpy as jnp
from jax import lax
from jax.experimental import pallas as pl
from jax.experimental.pallas import tpu as pltpu
from jax.experimental.pallas import tpu_sc as plsc

N = 10000
E = 320000
D = 128
QD = 32
EPT = E // 16
CH = 80
NCH = EPT // CH
NR = 10240
RPS = NR // 16
BR = 2000
GRID = N // BR

_NEG = -3.0e38


def _dense_body(with_bn, *refs):
    if with_bn:
        (x_ref, sum_ref, sq_ref, gam_ref, bet_ref, w_ref, as_ref, ad_ref,
         h0_ref, h1_ref, h2_ref, h3_ref, als_ref, ald_ref, mxs_ref,
         mxd_ref) = refs
        mu = sum_ref[...] * (1.0 / N)
        var = sq_ref[...] * (1.0 / N) - mu * mu
        rstd = 1.0 / jnp.sqrt(var + 1e-5)
        xb = x_ref[...]
        gam = gam_ref[...].reshape(1, D)
        bet = bet_ref[...].reshape(1, D)
        xb = gam * (xb - mu) * rstd + bet
        xb = jnp.maximum(xb, 0.0)
    else:
        (x_ref, w_ref, as_ref, ad_ref,
         h0_ref, h1_ref, h2_ref, h3_ref, als_ref, ald_ref, mxs_ref,
         mxd_ref) = refs
        xb = x_ref[...]
    i = pl.program_id(0)
    h = jnp.dot(xb, w_ref[...], preferred_element_type=jnp.float32)
    h0_ref[...] = h[:, 0 * QD:1 * QD]
    h1_ref[...] = h[:, 1 * QD:2 * QD]
    h2_ref[...] = h[:, 2 * QD:3 * QD]
    h3_ref[...] = h[:, 3 * QD:4 * QD]
    als = jnp.sum(h * as_ref[...], axis=1, keepdims=True)
    ald = jnp.sum(h * ad_ref[...], axis=1, keepdims=True)
    als_ref[...] = als
    ald_ref[...] = ald

    @pl.when(i == 0)
    def _():
        mxs_ref[...] = jnp.full((1, D), _NEG, jnp.float32)
        mxd_ref[...] = jnp.full((1, D), _NEG, jnp.float32)

    mxs_ref[...] = jnp.maximum(mxs_ref[...], jnp.max(als))
    mxd_ref[...] = jnp.maximum(mxd_ref[...], jnp.max(ald))


def _dense_call(with_bn, x, W, a_s, a_d, bn=None):
    row_spec = pl.BlockSpec((BR, D), lambda i: (i, 0))
    q_spec = pl.BlockSpec((BR, QD), lambda i: (i, 0))
    full_spec = pl.BlockSpec((1, D), lambda i: (0, 0))
    vec_spec = pl.BlockSpec((D,), lambda i: (0,))
    in_specs = [row_spec]
    args = [x]
    if with_bn:
        ssum, ssq, gam, bet = bn
        in_specs += [full_spec, full_spec, vec_spec, vec_spec]
        args += [ssum, ssq, gam, bet]
    in_specs += [pl.BlockSpec((D, D), lambda i: (0, 0)), full_spec, full_spec]
    args += [W, a_s, a_d]
    return pl.pallas_call(
        functools.partial(_dense_body, with_bn),
        grid=(GRID,),
        in_specs=in_specs,
        out_specs=[
            q_spec, q_spec, q_spec, q_spec,
            pl.BlockSpec((BR, 1), lambda i: (i, 0)),
            pl.BlockSpec((BR, 1), lambda i: (i, 0)),
            full_spec,
            full_spec,
        ],
        out_shape=[
            jax.ShapeDtypeStruct((N, QD), jnp.float32),
            jax.ShapeDtypeStruct((N, QD), jnp.float32),
            jax.ShapeDtypeStruct((N, QD), jnp.float32),
            jax.ShapeDtypeStruct((N, QD), jnp.float32),
            jax.ShapeDtypeStruct((N, 1), jnp.float32),
            jax.ShapeDtypeStruct((N, 1), jnp.float32),
            jax.ShapeDtypeStruct((1, D), jnp.float32),
            jax.ShapeDtypeStruct((1, D), jnp.float32),
        ],
    )(*args)


_sc_mesh = plsc.VectorSubcoreMesh(core_axis_name="c", subcore_axis_name="s")


@functools.partial(
    pl.kernel,
    mesh=_sc_mesh,
    compiler_params=pltpu.CompilerParams(needs_layout_passes=False,
                                         use_tc_tiling_on_sc=False),
    out_type=[
        jax.ShapeDtypeStruct((4, NR, QD), jnp.float32),
        jax.ShapeDtypeStruct((NR,), jnp.float32),
    ],
    scratch_types=[
        pltpu.VMEM((NCH, CH), jnp.int32),
        pltpu.VMEM((NCH, CH), jnp.int32),
        pltpu.VMEM((N,), jnp.float32),
        pltpu.VMEM((N,), jnp.float32),
        pltpu.VMEM((16,), jnp.float32),
        pltpu.VMEM((16,), jnp.float32),
        pltpu.VMEM((NCH, CH), jnp.float32),
        pltpu.VMEM((CH, QD), jnp.float32),
        pltpu.VMEM((CH, QD), jnp.float32),
        pltpu.VMEM((CH, QD), jnp.float32),
        pltpu.VMEM((CH, QD), jnp.float32),
        pltpu.VMEM((CH, QD), jnp.float32),
        pltpu.VMEM((RPS // 5, QD), jnp.float32),
        pltpu.VMEM((RPS,), jnp.float32),
        pltpu.VMEM_SHARED((NR, QD), jnp.float32),
        pltpu.VMEM_SHARED((NR,), jnp.float32),
        pltpu.SemaphoreType.DMA,
        pltpu.SemaphoreType.DMA,
        pltpu.SemaphoreType.DMA,
        pltpu.SemaphoreType.DMA,
        pltpu.SemaphoreType.DMA,
        pltpu.SemaphoreType.DMA,
        pltpu.SemaphoreType.DMA,
        pltpu.SemaphoreType.DMA,
        pltpu.SemaphoreType.DMA,
        pltpu.SemaphoreType.DMA,
        pltpu.SemaphoreType.DMA,
    ],
)
def _edge_kernel(edge_hbm, als_hbm, ald_hbm, mxs_hbm, mxd_hbm, h0_hbm,
                 h1_hbm, h2_hbm, h3_hbm, out_hbm, den_hbm,
                 src_v, dst_v, als_v, ald_v, mxs_v, mxd_v, w_v,
                 rows0, rows1, rows2, rows3, rows4, zrow_v, zden_v,
                 out_sh, den_sh,
                 gs0, gs1, gs2, gs3, gs4, ss0, ss1, ss2, ss3, ss4, dsem):
    c = lax.axis_index("c")
    s = lax.axis_index("s")
    rows = [rows0, rows1, rows2, rows3, rows4]
    gsem = [gs0, gs1, gs2, gs3, gs4]
    ssem = [ss0, ss1, ss2, ss3, ss4]

    pltpu.sync_copy(edge_hbm.at[0, s], src_v)
    pltpu.sync_copy(edge_hbm.at[1, s], dst_v)
    pltpu.sync_copy(als_hbm, als_v)
    pltpu.sync_copy(ald_hbm, ald_v)
    pltpu.sync_copy(mxs_hbm.at[0, pl.ds(0, 16)], mxs_v)
    pltpu.sync_copy(mxd_hbm.at[0, pl.ds(0, 16)], mxd_v)

    zv = jnp.zeros((16,), jnp.float32)

    def _zrow(r, carry):
        for t in range(QD // 16):
            zrow_v[r, pl.ds(t * 16, 16)] = zv
        return carry

    lax.fori_loop(0, RPS // 5, _zrow, 0)
    for g in range(RPS // 16):
        zden_v[pl.ds(g * 16, 16)] = zv

    def _zero_stripe():
        for q in range(5):
            pltpu.sync_copy(
                zrow_v, out_sh.at[pl.ds(s * RPS + q * (RPS // 5), RPS // 5)])

    _zero_stripe()
    pltpu.sync_copy(zden_v, den_sh.at[pl.ds(s * RPS, RPS)])
    plsc.subcore_barrier()

    mb = mxs_v[...] + mxd_v[...]
    m16 = jnp.where(mb > 0.0, mb, 0.2 * mb)

    def _wloop(j, carry):
        for v in range(CH // 16):
            si = src_v[j, pl.ds(v * 16, 16)]
            di = dst_v[j, pl.ds(v * 16, 16)]
            a1 = plsc.load_gather(als_v, [si])
            a2 = plsc.load_gather(ald_v, [di])
            z = a1 + a2
            z = jnp.where(z > 0.0, z, 0.2 * z)
            w_v[j, pl.ds(v * 16, 16)] = jnp.exp(z - m16)
        pltpu.async_copy(w_v.at[j], den_sh.at[dst_v.at[j]], dsem, add=True)
        return carry

    lax.fori_loop(0, NCH, _wloop, 0)

    def _issue_gather(j, b, first_pass):
        @pl.when(c == 0)
        def _():
            tab = h0_hbm if first_pass else h1_hbm
            pltpu.async_copy(tab.at[src_v.at[j]], rows[b], gsem[b])

        @pl.when(c == 1)
        def _():
            tab = h2_hbm if first_pass else h3_hbm
            pltpu.async_copy(tab.at[src_v.at[j]], rows[b], gsem[b])

    def _gather_wait(j, b, first_pass):
        @pl.when(c == 0)
        def _():
            tab = h0_hbm if first_pass else h1_hbm
            pltpu.make_async_copy(tab.at[src_v.at[j]], rows[b],
                                  gsem[b]).wait()

        @pl.when(c == 1)
        def _():
            tab = h2_hbm if first_pass else h3_hbm
            pltpu.make_async_copy(tab.at[src_v.at[j]], rows[b],
                                  gsem[b]).wait()

    def _scatter_wait(j, b):
        pltpu.make_async_copy(rows[b], out_sh.at[dst_v.at[j]],
                              ssem[b]).wait()

    def _scale(j, b):
        for v in range(CH // 16):
            wg = w_v[j, pl.ds(v * 16, 16)]
            for i in range(16):
                ws = wg[i]
                r = v * 16 + i
                for t in range(QD // 16):
                    rows[b][r, pl.ds(t * 16, 16)] = (
                        rows[b][r, pl.ds(t * 16, 16)] * ws)

    def _run_pass(first_pass):
        for b in range(3):
            _issue_gather(b, b, first_pass)

        def _outer(t, carry):
            for b in range(5):
                j = t * 5 + b
                _gather_wait(j, b, first_pass)
                _scale(j, b)
                pltpu.async_copy(rows[b], out_sh.at[dst_v.at[j]], ssem[b],
                                 add=True)
                b3 = (b + 3) % 5

                @pl.when(j + 3 < NCH)
                def _():
                    @pl.when(j >= 2)
                    def _():
                        _scatter_wait(j - 2, b3)
                    _issue_gather(j + 3, b3, first_pass)
            return carry

        lax.fori_loop(0, NCH // 5, _outer, 0)
        for b in range(5):
            _scatter_wait(NCH - 5 + b, b)

    _run_pass(True)
    def _den_drain(j, carry):
        pltpu.make_async_copy(w_v.at[j], den_sh.at[dst_v.at[j]], dsem).wait()
        return carry

    lax.fori_loop(0, NCH, _den_drain, 0)
    plsc.subcore_barrier()
    for qq in range(2):
        @pl.when(c == qq)
        def _():
            pltpu.sync_copy(out_sh.at[pl.ds(s * RPS, RPS)],
                            out_hbm.at[2 * qq, pl.ds(s * RPS, RPS)])
    @pl.when(c == 0)
    def _():
        pltpu.sync_copy(den_sh.at[pl.ds(s * RPS, RPS)],
                        den_hbm.at[pl.ds(s * RPS, RPS)])
    _zero_stripe()
    plsc.subcore_barrier()

    _run_pass(False)
    plsc.subcore_barrier()
    for qq in range(2):
        @pl.when(c == qq)
        def _():
            pltpu.sync_copy(out_sh.at[pl.ds(s * RPS, RPS)],
                            out_hbm.at[2 * qq + 1, pl.ds(s * RPS, RPS)])


def _norm_body(p_ref, d_ref, h0_ref, h1_ref, h2_ref, h3_ref, als_ref,
               ald_ref, mxs_ref, mxd_ref, b_ref, g_ref, sum_ref, sq_ref):
    i = pl.program_id(0)
    mb = mxs_ref[0, 0] + mxd_ref[0, 0]
    m = jnp.where(mb > 0.0, mb, 0.2 * mb)
    z = als_ref[...] + ald_ref[...]
    z = jnp.where(z > 0.0, z, 0.2 * z)
    ws = jnp.exp(z - m)
    h = jnp.concatenate(
        [h0_ref[...], h1_ref[...], h2_ref[...], h3_ref[...]], axis=1)
    p = jnp.concatenate(
        [p_ref[0], p_ref[1], p_ref[2], p_ref[3]], axis=1)
    num = p + ws * h
    den = d_ref[...] + ws
    g = num / (den + 1e-16) + b_ref[...].reshape(1, D)
    g_ref[...] = g

    @pl.when(i == 0)
    def _():
        sum_ref[...] = jnp.zeros((1, D), jnp.float32)
        sq_ref[...] = jnp.zeros((1, D), jnp.float32)

    sum_ref[...] += jnp.sum(g, axis=0, keepdims=True)
    sq_ref[...] += jnp.sum(g * g, axis=0, keepdims=True)


def _norm_call(p, d2, hq, als, ald, mxs, mxd, bias):
    return pl.pallas_call(
        _norm_body,
        grid=(GRID,),
        in_specs=[
            pl.BlockSpec((4, BR, QD), lambda i: (0, i, 0)),
            pl.BlockSpec((BR, 1), lambda i: (i, 0)),
            pl.BlockSpec((BR, QD), lambda i: (i, 0)),
            pl.BlockSpec((BR, QD), lambda i: (i, 0)),
            pl.BlockSpec((BR, QD), lambda i: (i, 0)),
            pl.BlockSpec((BR, QD), lambda i: (i, 0)),
            pl.BlockSpec((BR, 1), lambda i: (i, 0)),
            pl.BlockSpec((BR, 1), lambda i: (i, 0)),
            pl.BlockSpec((1, D), lambda i: (0, 0)),
            pl.BlockSpec((1, D), lambda i: (0, 0)),
            pl.BlockSpec((D,), lambda i: (0,)),
        ],
        out_specs=[
            pl.BlockSpec((BR, D), lambda i: (i, 0)),
            pl.BlockSpec((1, D), lambda i: (0, 0)),
            pl.BlockSpec((1, D), lambda i: (0, 0)),
        ],
        out_shape=[
            jax.ShapeDtypeStruct((N, D), jnp.float32),
            jax.ShapeDtypeStruct((1, D), jnp.float32),
            jax.ShapeDtypeStruct((1, D), jnp.float32),
        ],
    )(p, d2, hq[0], hq[1], hq[2], hq[3], als, ald, mxs, mxd, bias)


def _head_body(g_ref, sum_ref, sq_ref, gam_ref, bet_ref, emb_ref,
               w1a_ref, w1b_ref, b1_ref, w2_ref, b2_ref, out_ref):
    mu = sum_ref[...] * (1.0 / N)
    var = sq_ref[...] * (1.0 / N) - mu * mu
    rstd = 1.0 / jnp.sqrt(var + 1e-5)
    gam = gam_ref[...].reshape(1, D)
    bet = bet_ref[...].reshape(1, D)
    y = gam * (g_ref[...] - mu) * rstd + bet
    y = jnp.maximum(y, 0.0)
    z = jnp.dot(y, w1a_ref[...], preferred_element_type=jnp.float32)
    z += jnp.dot(emb_ref[...], w1b_ref[...], preferred_element_type=jnp.float32)
    z = jnp.maximum(z + b1_ref[...].reshape(1, D), 0.0)
    out_ref[...] = jnp.dot(z, w2_ref[...],
                           preferred_element_type=jnp.float32) \
        + b2_ref[...].reshape(1, 2)


def _head_call(g, ssum, ssq, gam, bet, emb, lin1_W, b1, w2, b2):
    full = pl.BlockSpec((1, D), lambda i: (0, 0))
    vec = pl.BlockSpec((D,), lambda i: (0,))
    return pl.pallas_call(
        _head_body,
        grid=(GRID,),
        in_specs=[
            pl.BlockSpec((BR, D), lambda i: (i, 0)),
            full, full, vec, vec,
            pl.BlockSpec((BR, 64), lambda i: (i, 0)),
            pl.BlockSpec((D, D), lambda i: (0, 0)),
            pl.BlockSpec((64, D), lambda i: (2, 0)),
            vec,
            pl.BlockSpec((D, 2), lambda i: (0, 0)),
            pl.BlockSpec((2,), lambda i: (0,)),
        ],
        out_specs=pl.BlockSpec((BR, 2), lambda i: (i, 0)),
        out_shape=jax.ShapeDtypeStruct((N, 2), jnp.float32),
    )(g, ssum, ssq, gam, bet, emb, lin1_W, lin1_W, b1, w2, b2)


def _gat_layer(edge_r, x, W, a_s, a_d, bias, bn):
    h0, h1, h2, h3, als, ald, mxs, mxd = _dense_call(bn is not None, x, W,
                                                     a_s, a_d, bn)
    outp, denp = _edge_kernel(edge_r, als.reshape(N), ald.reshape(N), mxs,
                              mxd, h0, h1, h2, h3)
    d2 = denp.reshape(NR, 1)
    g, ssum, ssq = _norm_call(outp, d2, (h0, h1, h2, h3), als, ald, mxs,
                              mxd, bias)
    return g, ssum, ssq


def kernel(x, edge_index, fair_node_embedding, W1, a_src1, a_dst1, b1,
           gamma1, beta1, W2, a_src2, a_dst2, b2, gamma2, beta2,
           lin1_W, lin1_b, lin2_W, lin2_b):
    edge_r = edge_index.reshape(2, 16, NCH, CH)

    g1, ssum1, ssq1 = _gat_layer(edge_r, x, W1, a_src1, a_dst1, b1, None)
    g2, ssum2, ssq2 = _gat_layer(edge_r, g1, W2, a_src2, a_dst2, b2,
                                 (ssum1, ssq1, gamma1, beta1))
    return _head_call(g2, ssum2, ssq2, gamma2, beta2,
                      fair_node_embedding, lin1_W, lin1_b, lin2_W, lin2_b)
